# Initial kernel scaffold; baseline (speedup 1.0000x reference)
import jax
import jax.numpy as jnp
from jax.experimental import pallas as pl


def kernel(x, edge_index, params):
    raise NotImplementedError("write your pallas kernel here")


# speedup vs baseline: 4.1599x; 4.1599x over previous
import functools

import jax
import jax.numpy as jnp
from jax import lax
from jax.experimental import pallas as pl
from jax.experimental.pallas import tpu as pltpu
from jax.experimental.pallas import tpu_sc as plsc

N = 10000
E = 320000
F = 128
NUM_LAYERS = 3
BN_EPS = 1e-5
BN_INV = 1.0 / (1.0 + BN_EPS) ** 0.5

NC = 2
NS = 16
NW = NC * NS
CHUNK = 128
NCHUNK = -(-E // (NW * CHUNK))
EPW = NCHUNK * CHUNK
EPAD = NW * EPW
AGG_ROWS = 10240
RPT = AGG_ROWS // NS
ZCOPIES = RPT // CHUNK


def _sc_agg_body(h_hbm, src_hbm, dst_hbm, out_hbm, src_v, dst_v, rows_v,
                 agg_sh, sem):
    c = lax.axis_index("c")
    s = lax.axis_index("s")
    wid = c * NS + s
    pltpu.sync_copy(src_hbm.at[wid], src_v)
    pltpu.sync_copy(dst_hbm.at[wid], dst_v)

    def zbody(i, _):
        rows_v[i // 8, pl.ds((i % 8) * 16, 16)] = jnp.zeros((16,), jnp.float32)
        return 0
    lax.fori_loop(0, CHUNK * 8, zbody, 0)
    base = s * RPT
    for k in range(ZCOPIES):
        pltpu.sync_copy(rows_v, agg_sh.at[pl.ds(base + k * CHUNK, CHUNK)])
    plsc.subcore_barrier()

    def body(j, _):
        pltpu.async_copy(h_hbm.at[src_v.at[j]], rows_v, sem).wait()
        pltpu.sync_copy(rows_v, agg_sh.at[dst_v.at[j]], add=True)
        return 0
    lax.fori_loop(0, NCHUNK, body, 0)
    plsc.subcore_barrier()

    pltpu.sync_copy(agg_sh.at[pl.ds(base, RPT)],
                    out_hbm.at[c, pl.ds(base, RPT)])


@functools.cache
def _sc_agg():
    return pl.kernel(
        _sc_agg_body,
        out_type=jax.ShapeDtypeStruct((NC, AGG_ROWS, F), jnp.float32),
        mesh=plsc.VectorSubcoreMesh(core_axis_name="c", subcore_axis_name="s",
                                    num_cores=NC, num_subcores=NS),
        scratch_types=[
            pltpu.VMEM((NCHUNK, CHUNK), jnp.int32),
            pltpu.VMEM((NCHUNK, CHUNK), jnp.int32),
            pltpu.VMEM((CHUNK, F), jnp.float32),
            pltpu.VMEM_SHARED((AGG_ROWS, F), jnp.float32),
            pltpu.SemaphoreType.DMA,
        ],
    )


def _mlp_body(relu_last, h_ref, a0_ref, a1_ref, wa_ref, ba_ref, ga_ref,
              bea_ref, wb_ref, bb_ref, gb_ref, beb_ref, go_ref, beo_ref,
              out_ref):
    m = h_ref[...] + a0_ref[...] + a1_ref[...]
    t = jnp.dot(m, wa_ref[...], preferred_element_type=jnp.float32)
    t = (t + ba_ref[...]) * (ga_ref[...] * BN_INV) + bea_ref[...]
    t = jnp.maximum(t, 0.0)
    t = jnp.dot(t, wb_ref[...], preferred_element_type=jnp.float32)
    t = (t + bb_ref[...]) * (gb_ref[...] * BN_INV) + beb_ref[...]
    if relu_last:
        t = jnp.maximum(t, 0.0)
    t = t * (go_ref[...] * BN_INV) + beo_ref[...]
    if relu_last:
        t = jnp.maximum(t, 0.0)
    out_ref[...] = t


BLK = 1000


def _mlp(h, a0, a1, p, l, last):
    vec = lambda v: v.reshape(1, F)
    args = (h, a0, a1,
            p['w%da' % l], vec(p['b%da' % l]), vec(p['g%da' % l]),
            vec(p['be%da' % l]),
            p['w%db' % l], vec(p['b%db' % l]), vec(p['g%db' % l]),
            vec(p['be%db' % l]),
            vec(p['g%do' % l]), vec(p['be%do' % l]))
    row_spec = pl.BlockSpec((BLK, F), lambda i: (i, 0))
    full = lambda a: pl.BlockSpec(a.shape, lambda i: (0, 0))
    return pl.pallas_call(
        functools.partial(_mlp_body, not last),
        grid=(N // BLK,),
        in_specs=[row_spec, row_spec, row_spec] + [full(a) for a in args[3:]],
        out_specs=row_spec,
        out_shape=jax.ShapeDtypeStruct((N, F), jnp.float32),
    )(*args)


def kernel(x, edge_index, params):
    src = edge_index[0].astype(jnp.int32)
    dst = edge_index[1].astype(jnp.int32)
    pad = EPAD - E
    src = jnp.concatenate([src, jnp.zeros((pad,), jnp.int32)])
    dst = jnp.concatenate([dst, jnp.full((pad,), N, jnp.int32)])
    src = src.reshape(NW, NCHUNK, CHUNK)
    dst = dst.reshape(NW, NCHUNK, CHUNK)
    h = x
    for l in range(NUM_LAYERS):
        aggs = _sc_agg()(h, src, dst)
        h = _mlp(h, aggs[0], aggs[1], params, l, last=(l == NUM_LAYERS - 1))
    return h
